# baseline (speedup 1.0000x reference)
import jax
import jax.numpy as jnp
from jax import lax
from jax.experimental import pallas as pl
from jax.experimental.pallas import tpu as pltpu
from jax.experimental.pallas import tpu_sc as plsc

N_CLASSES_ = 1000
N_REGIMES_ = 16
N_EXCIPIENTS_ = 100
B_ = 16384
T_ = 200

_NC = 2
_NS = 16
_NW = _NC * _NS
_L = 16
_BPW = B_ // _NW
_NROW = N_CLASSES_ * N_REGIMES_


_RH = 512
_RLAST = _NROW - _RH


def _sc_relayout_body(t0, t1, t2, t3, t4, o0, o1, o2, o3, o4, stage_a, stage_b, sem):
    c = lax.axis_index("c")
    s = lax.axis_index("s")
    wid = s * _NC + c
    start = jnp.minimum(wid * _RH, _RLAST)
    span = pl.ds(start, _RH)
    pairs = ((t0, o0), (t1, o1), (t2, o2), (t3, o3), (t4, o4))
    for i, (tbl, out) in enumerate(pairs):
        stage = stage_a if i % 2 == 0 else stage_b
        pltpu.sync_copy(tbl.at[span], stage)
        pltpu.sync_copy(stage, out.at[span])


def _sc_relayout(ss_t, dl_t, wb_t, wa_t, wl_t):
    out_sd = jax.ShapeDtypeStruct((_NROW, N_EXCIPIENTS_), jnp.float32)
    mesh = plsc.VectorSubcoreMesh(core_axis_name="c", subcore_axis_name="s")
    stg = pltpu.VMEM((_RH, N_EXCIPIENTS_), jnp.float32)
    fn = pl.kernel(
        _sc_relayout_body,
        mesh=mesh,
        out_type=(out_sd,) * 5,
        scratch_types=[stg, stg, pltpu.SemaphoreType.DMA],
    )
    return fn(ss_t, dl_t, wb_t, wa_t, wl_t)


def _sc_gather_body(p_hbm, r_hbm, e_hbm, ss_hbm, dl_hbm, wb_hbm, wa_hbm,
                    wl_hbm, th_hbm,
                    a_out, it_out, wb_out, wa_out, wl_out,
                    pv, rv, ev, fv,
                    g_ss, g_dl, g_wb, g_wa, g_wl, g_th,
                    o_a, o_it, sem):
    c = lax.axis_index("c")
    s = lax.axis_index("s")
    wid = s * _NC + c
    rows = pl.ds(wid * _BPW, _BPW)
    pltpu.sync_copy(p_hbm.at[rows], pv)
    pltpu.sync_copy(r_hbm.at[rows], rv)
    pltpu.sync_copy(e_hbm.at[rows], ev)

    for k in range(_BPW // _L):
        sl = pl.ds(k * _L, _L)
        fv[sl] = (pv[sl] * N_REGIMES_ + rv[sl]) * N_EXCIPIENTS_ + ev[sl]

    handles = [
        pltpu.async_copy(ss_hbm.at[fv], g_ss, sem),
        pltpu.async_copy(dl_hbm.at[fv], g_dl, sem),
        pltpu.async_copy(wb_hbm.at[fv], g_wb, sem),
        pltpu.async_copy(wa_hbm.at[fv], g_wa, sem),
        pltpu.async_copy(wl_hbm.at[fv], g_wl, sem),
        pltpu.async_copy(th_hbm.at[ev], g_th, sem),
    ]
    for h in handles:
        h.wait()

    for k in range(_BPW // _L):
        sl = pl.ds(k * _L, _L)
        d = jnp.minimum(jnp.maximum(g_dl[sl], -5.0), 5.0)
        o_a[sl] = g_ss[sl] + d
        t = jnp.maximum(jnp.abs(g_th[sl]), 0.1)
        o_it[sl] = 1.0 / t

    pltpu.sync_copy(o_a, a_out.at[rows])
    pltpu.sync_copy(o_it, it_out.at[rows])
    pltpu.sync_copy(g_wb, wb_out.at[rows])
    pltpu.sync_copy(g_wa, wa_out.at[rows])
    pltpu.sync_copy(g_wl, wl_out.at[rows])


def _sc_gather(p1, r1, e1, ss_f, dl_f, wb_f, wa_f, wl_f, th):
    out_sd = jax.ShapeDtypeStruct((B_,), jnp.float32)
    mesh = plsc.VectorSubcoreMesh(core_axis_name="c", subcore_axis_name="s")
    vm_i = pltpu.VMEM((_BPW,), jnp.int32)
    vm_f = pltpu.VMEM((_BPW,), jnp.float32)
    fn = pl.kernel(
        _sc_gather_body,
        mesh=mesh,
        out_type=(out_sd,) * 5,
        scratch_types=[vm_i, vm_i, vm_i, vm_i,
                       vm_f, vm_f, vm_f, vm_f, vm_f, vm_f,
                       vm_f, vm_f,
                       pltpu.SemaphoreType.DMA],
    )
    return fn(p1, r1, e1, ss_f, dl_f, wb_f, wa_f, wl_f, th)


_BR = 2048


def _tc_body(sharp_ref, raw_ref, a_ref, it_ref, wb_ref, wa_ref, wl_ref,
             res_ref, gate_ref, conc_ref):
    s = jnp.clip(sharp_ref[0, 0], 1.0, 20.0)
    x = raw_ref[...]
    sp = jnp.maximum(x, 0.0) + jnp.log1p(jnp.exp(-jnp.abs(x)))
    cr = sp * it_ref[...]
    gate = 1.0 / (1.0 + jnp.exp(-(s * (cr - 1.0))))
    effect_below = jnp.tanh(cr) * wb_ref[...]
    effect_above = jnp.log1p(cr) * wa_ref[...]
    conc = (1.0 - gate) * effect_below + gate * effect_above + cr * wl_ref[...]
    res_ref[...] = a_ref[...] * conc
    gate_ref[...] = gate
    conc_ref[...] = conc


def _tc_combine(raw, a_col, it_col, wb_col, wa_col, wl_col, sharp11):
    grid = (B_ // _BR,)
    col_spec = pl.BlockSpec((_BR, 1), lambda i: (i, 0))
    out_sd = jax.ShapeDtypeStruct((B_, T_), jnp.float32)
    return pl.pallas_call(
        _tc_body,
        grid=grid,
        in_specs=[
            pl.BlockSpec((1, 1), lambda i: (0, 0)),
            pl.BlockSpec((_BR, T_), lambda i: (i, 0)),
            col_spec, col_spec, col_spec, col_spec, col_spec,
        ],
        out_specs=[pl.BlockSpec((_BR, T_), lambda i: (i, 0))] * 3,
        out_shape=[out_sd] * 3,
    )(sharp11, raw, a_col, it_col, wb_col, wa_col, wl_col)


def kernel(p_idx, r_idx, e_idx, raw_concentration, static_scores, delta,
           thresholds, w_below, w_above, w_linear, sharpness):
    p1 = p_idx.astype(jnp.int32)
    r1 = r_idx.astype(jnp.int32)
    e1 = e_idx.astype(jnp.int32)
    n_tab = _NROW * N_EXCIPIENTS_
    ss_c, dl_c, wb_c, wa_c, wl_c = _sc_relayout(
        static_scores.reshape(_NROW, N_EXCIPIENTS_),
        delta.reshape(_NROW, N_EXCIPIENTS_),
        w_below.reshape(_NROW, N_EXCIPIENTS_),
        w_above.reshape(_NROW, N_EXCIPIENTS_),
        w_linear.reshape(_NROW, N_EXCIPIENTS_))
    ss_f = ss_c.reshape(n_tab)
    dl_f = dl_c.reshape(n_tab)
    wb_f = wb_c.reshape(n_tab)
    wa_f = wa_c.reshape(n_tab)
    wl_f = wl_c.reshape(n_tab)

    a1, it1, wb1, wa1, wl1 = _sc_gather(p1, r1, e1, ss_f, dl_f, wb_f, wa_f,
                                        wl_f, thresholds)

    a_col = a1.reshape(B_, 1)
    it_col = it1.reshape(B_, 1)
    wb_col = wb1.reshape(B_, 1)
    wa_col = wa1.reshape(B_, 1)
    wl_col = wl1.reshape(B_, 1)
    sharp11 = sharpness.astype(jnp.float32).reshape(1, 1)

    result, gate, conc_term = _tc_combine(raw_concentration, a_col, it_col,
                                          wb_col, wa_col, wl_col, sharp11)
    return (result, gate, conc_term)


# speedup vs baseline: 1.2205x; 1.2205x over previous
import jax
import jax.numpy as jnp
from jax import lax
from jax.experimental import pallas as pl
from jax.experimental.pallas import tpu as pltpu
from jax.experimental.pallas import tpu_sc as plsc

N_CLASSES_ = 1000
N_REGIMES_ = 16
N_EXCIPIENTS_ = 100
B_ = 16384
T_ = 200

_NC = 2
_NS = 16
_NW = _NC * _NS
_L = 16
_BPW = B_ // _NW
_NROW = N_CLASSES_ * N_REGIMES_


def _sc_gather_body(p_hbm, r_hbm, e_hbm, ss_hbm, dl_hbm, wb_hbm, wa_hbm,
                    wl_hbm, th_hbm,
                    a_out, it_out, wb_out, wa_out, wl_out,
                    pv, rv, ev, fv,
                    g_ss, g_dl, g_wb, g_wa, g_wl, g_th,
                    o_a, o_it, sem):
    c = lax.axis_index("c")
    s = lax.axis_index("s")
    wid = s * _NC + c
    rows = pl.ds(wid * _BPW, _BPW)
    pltpu.sync_copy(p_hbm.at[rows], pv)
    pltpu.sync_copy(r_hbm.at[rows], rv)
    pltpu.sync_copy(e_hbm.at[rows], ev)

    for k in range(_BPW // _L):
        sl = pl.ds(k * _L, _L)
        fv[sl] = (pv[sl] * N_REGIMES_ + rv[sl]) * N_EXCIPIENTS_ + ev[sl]

    handles = [
        pltpu.async_copy(ss_hbm.at[fv], g_ss, sem),
        pltpu.async_copy(dl_hbm.at[fv], g_dl, sem),
        pltpu.async_copy(wb_hbm.at[fv], g_wb, sem),
        pltpu.async_copy(wa_hbm.at[fv], g_wa, sem),
        pltpu.async_copy(wl_hbm.at[fv], g_wl, sem),
        pltpu.async_copy(th_hbm.at[ev], g_th, sem),
    ]
    for h in handles:
        h.wait()

    for k in range(_BPW // _L):
        sl = pl.ds(k * _L, _L)
        d = jnp.minimum(jnp.maximum(g_dl[sl], -5.0), 5.0)
        o_a[sl] = g_ss[sl] + d
        t = jnp.maximum(jnp.abs(g_th[sl]), 0.1)
        o_it[sl] = 1.0 / t

    pltpu.sync_copy(o_a, a_out.at[rows])
    pltpu.sync_copy(o_it, it_out.at[rows])
    pltpu.sync_copy(g_wb, wb_out.at[rows])
    pltpu.sync_copy(g_wa, wa_out.at[rows])
    pltpu.sync_copy(g_wl, wl_out.at[rows])


def _sc_gather(p1, r1, e1, ss_f, dl_f, wb_f, wa_f, wl_f, th):
    out_sd = jax.ShapeDtypeStruct((B_,), jnp.float32)
    mesh = plsc.VectorSubcoreMesh(core_axis_name="c", subcore_axis_name="s")
    vm_i = pltpu.VMEM((_BPW,), jnp.int32)
    vm_f = pltpu.VMEM((_BPW,), jnp.float32)
    fn = pl.kernel(
        _sc_gather_body,
        mesh=mesh,
        out_type=(out_sd,) * 5,
        scratch_types=[vm_i, vm_i, vm_i, vm_i,
                       vm_f, vm_f, vm_f, vm_f, vm_f, vm_f,
                       vm_f, vm_f,
                       pltpu.SemaphoreType.DMA],
    )
    return fn(p1, r1, e1, ss_f, dl_f, wb_f, wa_f, wl_f, th)


_BR = 2048


def _tc_body(sharp_ref, raw_ref, a_ref, it_ref, wb_ref, wa_ref, wl_ref,
             res_ref, gate_ref, conc_ref):
    s = jnp.clip(sharp_ref[0, 0], 1.0, 20.0)
    x = raw_ref[...]
    sp = jnp.maximum(x, 0.0) + jnp.log1p(jnp.exp(-jnp.abs(x)))
    cr = sp * it_ref[...]
    gate = 1.0 / (1.0 + jnp.exp(-(s * (cr - 1.0))))
    effect_below = jnp.tanh(cr) * wb_ref[...]
    effect_above = jnp.log1p(cr) * wa_ref[...]
    conc = (1.0 - gate) * effect_below + gate * effect_above + cr * wl_ref[...]
    res_ref[...] = a_ref[...] * conc
    gate_ref[...] = gate
    conc_ref[...] = conc


def _tc_combine(raw, a_col, it_col, wb_col, wa_col, wl_col, sharp11):
    grid = (B_ // _BR,)
    col_spec = pl.BlockSpec((_BR, 1), lambda i: (i, 0))
    out_sd = jax.ShapeDtypeStruct((B_, T_), jnp.float32)
    return pl.pallas_call(
        _tc_body,
        grid=grid,
        in_specs=[
            pl.BlockSpec((1, 1), lambda i: (0, 0)),
            pl.BlockSpec((_BR, T_), lambda i: (i, 0)),
            col_spec, col_spec, col_spec, col_spec, col_spec,
        ],
        out_specs=[pl.BlockSpec((_BR, T_), lambda i: (i, 0))] * 3,
        out_shape=[out_sd] * 3,
    )(sharp11, raw, a_col, it_col, wb_col, wa_col, wl_col)


def kernel(p_idx, r_idx, e_idx, raw_concentration, static_scores, delta,
           thresholds, w_below, w_above, w_linear, sharpness):
    p1 = p_idx.astype(jnp.int32)
    r1 = r_idx.astype(jnp.int32)
    e1 = e_idx.astype(jnp.int32)
    n_tab = _NROW * N_EXCIPIENTS_
    ss_f = static_scores.reshape(n_tab)
    dl_f = delta.reshape(n_tab)
    wb_f = w_below.reshape(n_tab)
    wa_f = w_above.reshape(n_tab)
    wl_f = w_linear.reshape(n_tab)

    a1, it1, wb1, wa1, wl1 = _sc_gather(p1, r1, e1, ss_f, dl_f, wb_f, wa_f,
                                        wl_f, thresholds)

    a_col = a1.reshape(B_, 1)
    it_col = it1.reshape(B_, 1)
    wb_col = wb1.reshape(B_, 1)
    wa_col = wa1.reshape(B_, 1)
    wl_col = wl1.reshape(B_, 1)
    sharp11 = sharpness.astype(jnp.float32).reshape(1, 1)

    result, gate, conc_term = _tc_combine(raw_concentration, a_col, it_col,
                                          wb_col, wa_col, wl_col, sharp11)
    return (result, gate, conc_term)
